# baseline (speedup 1.0000x reference)
import jax
import jax.numpy as jnp
from jax import lax
from jax.experimental import pallas as pl
from jax.experimental.pallas import tpu as pltpu
from jax.experimental.pallas import tpu_sc as plsc

N = 10000
E = 320000
NF = 128
EF = 16
MSG = 128
H = 300
NH = 128
NP = 32
G = 64

NC = 2
NS = 16
NW = NC * NS

ECH = 128
CPW = 80
EPAD = NW * CPW * ECH

NPAD = 10240
STRIPE = NPAD // NS
DUMP = NPAD - 1

_MESH = dict(core_axis_name="c", subcore_axis_name="s", num_cores=NC,
             num_subcores=NS)


def _gather_body(x_hbm, src_hbm, dst_hbm, xi_hbm, xj_hbm,
                 idx_s, idx_d, bi0, bi1, bj0, bj1, sem):
  wid = lax.axis_index("s") * NC + lax.axis_index("c")
  pltpu.sync_copy(src_hbm.at[wid], idx_s)
  pltpu.sync_copy(dst_hbm.at[wid], idx_d)
  base = wid * CPW

  def step(m, carry):
    r0 = 2 * m
    r1 = 2 * m + 1
    c0 = pltpu.async_copy(x_hbm.at[idx_d.at[r0]], bi0, sem)
    c1 = pltpu.async_copy(x_hbm.at[idx_s.at[r0]], bj0, sem)
    c2 = pltpu.async_copy(x_hbm.at[idx_d.at[r1]], bi1, sem)
    c3 = pltpu.async_copy(x_hbm.at[idx_s.at[r1]], bj1, sem)
    c0.wait()
    c1.wait()
    c2.wait()
    c3.wait()
    off = (base + r0) * ECH
    pltpu.sync_copy(bi0, xi_hbm.at[pl.ds(off, ECH)])
    pltpu.sync_copy(bj0, xj_hbm.at[pl.ds(off, ECH)])
    pltpu.sync_copy(bi1, xi_hbm.at[pl.ds(off + ECH, ECH)])
    pltpu.sync_copy(bj1, xj_hbm.at[pl.ds(off + ECH, ECH)])
    return carry

  lax.fori_loop(0, CPW // 2, step, 0)


def _sc_gather(x, src3, dst3):
  mesh = plsc.VectorSubcoreMesh(**_MESH)
  buf = pltpu.VMEM((ECH, NF), jnp.float32)
  fn = pl.kernel(
      _gather_body,
      out_type=(jax.ShapeDtypeStruct((EPAD, NF), jnp.float32),
                jax.ShapeDtypeStruct((EPAD, NF), jnp.float32)),
      mesh=mesh,
      scratch_types=[
          pltpu.VMEM((CPW, ECH), jnp.int32),
          pltpu.VMEM((CPW, ECH), jnp.int32),
          buf, buf, buf, buf,
          pltpu.SemaphoreType.DMA,
      ],
  )
  return fn(x, src3, dst3)


def _scatter_body(msg_hbm, dst_hbm, zero_hbm, out_hbm,
                  idx_d, mb0, mb1, acc, sem_m):
  cid = lax.axis_index("c")
  sid = lax.axis_index("s")
  wid = sid * NC + cid
  pltpu.sync_copy(zero_hbm.at[pl.ds(sid * STRIPE, STRIPE)],
                  acc.at[pl.ds(sid * STRIPE, STRIPE)])
  plsc.subcore_barrier()

  pltpu.sync_copy(dst_hbm.at[wid], idx_d)
  mbs = [mb0, mb1]
  base = wid * CPW

  def load(r, slot):
    off = (base + r) * ECH
    pltpu.async_copy(msg_hbm.at[pl.ds(off, ECH)], mbs[slot], sem_m)

  def wait_load(slot):
    pltpu.make_async_copy(msg_hbm.at[pl.ds(0, ECH)], mbs[slot], sem_m).wait()

  load(0, 0)

  def step(m, carry):
    for b in range(2):
      r = 2 * m + b

      @pl.when(r < CPW)
      def _():
        @pl.when(r + 1 < CPW)
        def _():
          load(r + 1, (b + 1) % 2)

        wait_load(b)
        pltpu.sync_copy(mbs[b], acc.at[idx_d.at[r]], add=True)
    return carry

  lax.fori_loop(0, (CPW + 1) // 2, step, 0)

  plsc.subcore_barrier()
  pltpu.sync_copy(acc.at[pl.ds(sid * STRIPE, STRIPE)],
                  out_hbm.at[cid, pl.ds(sid * STRIPE, STRIPE)])


def _sc_scatter(msg, dst3, zero):
  mesh = plsc.VectorSubcoreMesh(**_MESH)
  fn = pl.kernel(
      _scatter_body,
      out_type=jax.ShapeDtypeStruct((NC, NPAD, MSG), jnp.float32),
      mesh=mesh,
      scratch_types=[
          pltpu.VMEM((CPW, ECH), jnp.int32),
          pltpu.VMEM((ECH, MSG), jnp.float32),
          pltpu.VMEM((ECH, MSG), jnp.float32),
          pltpu.VMEM_SHARED((NPAD, MSG), jnp.float32),
          pltpu.SemaphoreType.DMA,
      ],
  )
  return fn(msg, dst3, zero)


BE = 4096
NEB = EPAD // BE


def _emlp_body(xi_ref, xj_ref, ea_ref, w1a, w1b, w1c, b1, w2, b2, w3, b3,
               w4, b4, out_ref):
  f32 = jnp.float32
  h = jnp.dot(xi_ref[...], w1a[...], preferred_element_type=f32)
  h = h + jnp.dot(xj_ref[...], w1b[...], preferred_element_type=f32)
  h = h + jnp.dot(ea_ref[...], w1c[...], preferred_element_type=f32)
  h = jnp.maximum(h + b1[...], 0.0)
  h = jnp.maximum(jnp.dot(h, w2[...], preferred_element_type=f32) + b2[...],
                  0.0)
  h = jnp.maximum(jnp.dot(h, w3[...], preferred_element_type=f32) + b3[...],
                  0.0)
  out_ref[...] = jnp.dot(h, w4[...], preferred_element_type=f32) + b4[...]


def _tc_edge_mlp(xi, xj, ea, w1a, w1b, w1c, b1, w2, b2, w3, b3, w4, b4):
  ws = lambda shape: pl.BlockSpec(shape, lambda i: (0, 0))
  return pl.pallas_call(
      _emlp_body,
      grid=(NEB,),
      in_specs=[
          pl.BlockSpec((BE, NF), lambda i: (i, 0)),
          pl.BlockSpec((BE, NF), lambda i: (i, 0)),
          pl.BlockSpec((BE, EF), lambda i: (i, 0)),
          ws((NF, H)), ws((NF, H)), ws((EF, H)), ws((1, H)),
          ws((H, H)), ws((1, H)),
          ws((H, H)), ws((1, H)),
          ws((H, MSG)), ws((1, MSG)),
      ],
      out_specs=pl.BlockSpec((BE, MSG), lambda i: (i, 0)),
      out_shape=jax.ShapeDtypeStruct((EPAD, MSG), jnp.float32),
      compiler_params=pltpu.CompilerParams(
          dimension_semantics=("arbitrary",)),
  )(xi, xj, ea, w1a, w1b, w1c, b1, w2, b2, w3, b3, w4, b4)


BN = 2000
NBLK = N // BN


def _node_body(parts_ref, batch_ref, w1, b1, w2, b2, w3, b3, w4, b4,
               lw, blr, out_ref, acc_s, acc_c):
  f32 = jnp.float32
  i = pl.program_id(0)

  @pl.when(i == 0)
  def _():
    acc_s[...] = jnp.zeros_like(acc_s)
    acc_c[...] = jnp.zeros_like(acc_c)

  aggr = parts_ref[0] + parts_ref[1]
  h = jnp.maximum(jnp.dot(aggr, w1[...], preferred_element_type=f32)
                  + b1[...], 0.0)
  h = jnp.maximum(jnp.dot(h, w2[...], preferred_element_type=f32) + b2[...],
                  0.0)
  h = jnp.maximum(jnp.dot(h, w3[...], preferred_element_type=f32) + b3[...],
                  0.0)
  node = jnp.dot(h, w4[...], preferred_element_type=f32) + b4[...]

  gid = lax.broadcasted_iota(jnp.int32, (BN, G), 1)
  oh = jnp.where(batch_ref[...] == gid, 1.0, 0.0).astype(f32)
  dn = (((0,), (0,)), ((), ()))
  acc_s[...] += lax.dot_general(oh, node, dn, preferred_element_type=f32)
  ones = jnp.ones((BN, MSG), f32)
  acc_c[...] += lax.dot_general(oh, ones, dn, preferred_element_type=f32)

  @pl.when(i == NBLK - 1)
  def _():
    pooled = acc_s[...] / jnp.maximum(acc_c[...], 1.0)
    out_ref[...] = (jnp.dot(pooled, lw[...], preferred_element_type=f32)
                    + blr[...])


def _tc_node(parts, batch_i, w1, b1, w2, b2, w3, b3, w4, b4, lw, bl):
  ws = lambda shape: pl.BlockSpec(shape, lambda i: (0, 0))
  return pl.pallas_call(
      _node_body,
      grid=(NBLK,),
      in_specs=[
          pl.BlockSpec((NC, BN, MSG), lambda i: (0, i, 0)),
          pl.BlockSpec((BN, 1), lambda i: (i, 0)),
          ws((MSG, H)), ws((1, H)),
          ws((H, H)), ws((1, H)),
          ws((H, H)), ws((1, H)),
          ws((H, NH)), ws((1, NH)),
          ws((NH, NP)), ws((1, NP)),
      ],
      out_specs=pl.BlockSpec((G, NP), lambda i: (0, 0)),
      out_shape=jax.ShapeDtypeStruct((G, NP), jnp.float32),
      scratch_shapes=[
          pltpu.VMEM((G, NH), jnp.float32),
          pltpu.VMEM((G, NH), jnp.float32),
      ],
      compiler_params=pltpu.CompilerParams(
          dimension_semantics=("arbitrary",)),
  )(parts, batch_i, w1, b1, w2, b2, w3, b3, w4, b4, lw, bl)


def kernel(x, edge_index, edge_attr, batch,
           mW1, mb1, mW2, mb2, mW3, mb3, mW4, mb4,
           nW1, nb1, nW2, nb2, nW3, nb3, nW4, nb4,
           L, bL):
  pad1 = ((0, EPAD - E),)
  src3 = jnp.pad(edge_index[0], pad1).reshape(NW, CPW, ECH)
  dst3g = jnp.pad(edge_index[1], pad1).reshape(NW, CPW, ECH)
  dst3s = jnp.pad(edge_index[1], pad1,
                  constant_values=DUMP).reshape(NW, CPW, ECH)
  ea_p = jnp.pad(edge_attr, ((0, EPAD - E), (0, 0)))

  xi, xj = _sc_gather(x, src3, dst3g)

  w1a = mW1[:NF]
  w1b = mW1[NF:2 * NF]
  w1c = mW1[2 * NF:]
  msg = _tc_edge_mlp(xi, xj, ea_p,
                     w1a, w1b, w1c, mb1.reshape(1, H),
                     mW2, mb2.reshape(1, H),
                     mW3, mb3.reshape(1, H),
                     mW4, mb4.reshape(1, MSG))

  zero = jnp.zeros((NPAD, MSG), jnp.float32)
  parts = _sc_scatter(msg, dst3s, zero)

  batch_i = batch.reshape(N, 1)
  out = _tc_node(parts, batch_i,
                 nW1, nb1.reshape(1, H),
                 nW2, nb2.reshape(1, H),
                 nW3, nb3.reshape(1, H),
                 nW4, nb4.reshape(1, NH),
                 L, bL.reshape(1, NP))
  return out


# speedup vs baseline: 1.7217x; 1.7217x over previous
import jax
import jax.numpy as jnp
from jax import lax
from jax.experimental import pallas as pl
from jax.experimental.pallas import tpu as pltpu
from jax.experimental.pallas import tpu_sc as plsc

N = 10000
E = 320000
NF = 128
EF = 16
MSG = 128
H = 300
NH = 128
NP = 32
G = 64

NC = 2
NS = 16
NW = NC * NS

ECH = 128
CPW = 80
EPAD = NW * CPW * ECH

NPAD = 10240
STRIPE = NPAD // NS
DUMP = NPAD - 1

_MESH = dict(core_axis_name="c", subcore_axis_name="s", num_cores=NC,
             num_subcores=NS)


def _gather_body(x_hbm, src_hbm, dst_hbm, xi_hbm, xj_hbm,
                 idx_s, idx_d, bi0, bi1, bj0, bj1, sem):
  wid = lax.axis_index("s") * NC + lax.axis_index("c")
  pltpu.sync_copy(src_hbm.at[wid], idx_s)
  pltpu.sync_copy(dst_hbm.at[wid], idx_d)
  base = wid * CPW

  def step(m, carry):
    r0 = 2 * m
    r1 = 2 * m + 1
    c0 = pltpu.async_copy(x_hbm.at[idx_d.at[r0]], bi0, sem)
    c1 = pltpu.async_copy(x_hbm.at[idx_s.at[r0]], bj0, sem)
    c2 = pltpu.async_copy(x_hbm.at[idx_d.at[r1]], bi1, sem)
    c3 = pltpu.async_copy(x_hbm.at[idx_s.at[r1]], bj1, sem)
    c0.wait()
    c1.wait()
    c2.wait()
    c3.wait()
    off = (base + r0) * ECH
    pltpu.sync_copy(bi0, xi_hbm.at[pl.ds(off, ECH)])
    pltpu.sync_copy(bj0, xj_hbm.at[pl.ds(off, ECH)])
    pltpu.sync_copy(bi1, xi_hbm.at[pl.ds(off + ECH, ECH)])
    pltpu.sync_copy(bj1, xj_hbm.at[pl.ds(off + ECH, ECH)])
    return carry

  lax.fori_loop(0, CPW // 2, step, 0)


def _sc_gather(x, src3, dst3):
  mesh = plsc.VectorSubcoreMesh(**_MESH)
  buf = pltpu.VMEM((ECH, NF), jnp.float32)
  fn = pl.kernel(
      _gather_body,
      out_type=(jax.ShapeDtypeStruct((EPAD, NF), jnp.float32),
                jax.ShapeDtypeStruct((EPAD, NF), jnp.float32)),
      mesh=mesh,
      scratch_types=[
          pltpu.VMEM((CPW, ECH), jnp.int32),
          pltpu.VMEM((CPW, ECH), jnp.int32),
          buf, buf, buf, buf,
          pltpu.SemaphoreType.DMA,
      ],
  )
  return fn(x, src3, dst3)


def _scatter_body(msg_hbm, dst_hbm, zero_hbm, out_hbm,
                  idx_d, mb0, mb1, acc, sem_m):
  cid = lax.axis_index("c")
  sid = lax.axis_index("s")
  wid = sid * NC + cid
  pltpu.sync_copy(zero_hbm.at[pl.ds(sid * STRIPE, STRIPE)],
                  acc.at[pl.ds(sid * STRIPE, STRIPE)])
  plsc.subcore_barrier()

  pltpu.sync_copy(dst_hbm.at[wid], idx_d)
  mbs = [mb0, mb1]
  base = wid * CPW

  def load(r, slot):
    off = (base + r) * ECH
    pltpu.async_copy(msg_hbm.at[pl.ds(off, ECH)], mbs[slot], sem_m)

  def wait_load(slot):
    pltpu.make_async_copy(msg_hbm.at[pl.ds(0, ECH)], mbs[slot], sem_m).wait()

  load(0, 0)

  def step(m, carry):
    for b in range(2):
      r = 2 * m + b

      @pl.when(r < CPW)
      def _():
        @pl.when(r + 1 < CPW)
        def _():
          load(r + 1, (b + 1) % 2)

        wait_load(b)
        pltpu.sync_copy(mbs[b], acc.at[idx_d.at[r]], add=True)
    return carry

  lax.fori_loop(0, (CPW + 1) // 2, step, 0)

  plsc.subcore_barrier()
  pltpu.sync_copy(acc.at[pl.ds(sid * STRIPE, STRIPE)],
                  out_hbm.at[cid, pl.ds(sid * STRIPE, STRIPE)])


def _sc_scatter(msg, dst3, zero):
  mesh = plsc.VectorSubcoreMesh(**_MESH)
  fn = pl.kernel(
      _scatter_body,
      out_type=jax.ShapeDtypeStruct((NC, NPAD, MSG), jnp.float32),
      mesh=mesh,
      scratch_types=[
          pltpu.VMEM((CPW, ECH), jnp.int32),
          pltpu.VMEM((ECH, MSG), jnp.float32),
          pltpu.VMEM((ECH, MSG), jnp.float32),
          pltpu.VMEM_SHARED((NPAD, MSG), jnp.float32),
          pltpu.SemaphoreType.DMA,
      ],
  )
  return fn(msg, dst3, zero)


BE = 4096
NEB = EPAD // BE


def _emlp_body(xi_ref, xj_ref, ea_ref, w1a, w1b, w1c, b1, w2, b2, w3, b3,
               w4, b4, out_ref):
  f32 = jnp.float32
  h = jnp.dot(xi_ref[...], w1a[...], preferred_element_type=f32)
  h = h + jnp.dot(xj_ref[...], w1b[...], preferred_element_type=f32)
  h = h + jnp.dot(ea_ref[...], w1c[...], preferred_element_type=f32)
  h = jnp.maximum(h + b1[...], 0.0)
  h = jnp.maximum(jnp.dot(h, w2[...], preferred_element_type=f32) + b2[...],
                  0.0)
  h = jnp.maximum(jnp.dot(h, w3[...], preferred_element_type=f32) + b3[...],
                  0.0)
  out_ref[...] = jnp.dot(h, w4[...], preferred_element_type=f32) + b4[...]


def _tc_edge_mlp(xi, xj, ea, w1a, w1b, w1c, b1, w2, b2, w3, b3, w4, b4):
  ws = lambda shape: pl.BlockSpec(shape, lambda i: (0, 0))
  return pl.pallas_call(
      _emlp_body,
      grid=(NEB,),
      in_specs=[
          pl.BlockSpec((BE, NF), lambda i: (i, 0)),
          pl.BlockSpec((BE, NF), lambda i: (i, 0)),
          pl.BlockSpec((BE, EF), lambda i: (i, 0)),
          ws((NF, H)), ws((NF, H)), ws((EF, H)), ws((1, H)),
          ws((H, H)), ws((1, H)),
          ws((H, H)), ws((1, H)),
          ws((H, MSG)), ws((1, MSG)),
      ],
      out_specs=pl.BlockSpec((BE, MSG), lambda i: (i, 0)),
      out_shape=jax.ShapeDtypeStruct((EPAD, MSG), jnp.float32),
      compiler_params=pltpu.CompilerParams(
          dimension_semantics=("arbitrary",)),
  )(xi, xj, ea, w1a, w1b, w1c, b1, w2, b2, w3, b3, w4, b4)


BN = 2000
NBLK = N // BN


def _node_body(parts_ref, batch_ref, w1, b1, w2, b2, w3, b3, w4, b4,
               lw, blr, out_ref, acc_s, acc_c):
  f32 = jnp.float32
  i = pl.program_id(0)

  @pl.when(i == 0)
  def _():
    acc_s[...] = jnp.zeros_like(acc_s)
    acc_c[...] = jnp.zeros_like(acc_c)

  aggr = parts_ref[0] + parts_ref[1]
  h = jnp.maximum(jnp.dot(aggr, w1[...], preferred_element_type=f32)
                  + b1[...], 0.0)
  h = jnp.maximum(jnp.dot(h, w2[...], preferred_element_type=f32) + b2[...],
                  0.0)
  h = jnp.maximum(jnp.dot(h, w3[...], preferred_element_type=f32) + b3[...],
                  0.0)
  node = jnp.dot(h, w4[...], preferred_element_type=f32) + b4[...]

  gid = lax.broadcasted_iota(jnp.int32, (BN, G), 1)
  oh = jnp.where(batch_ref[...] == gid, 1.0, 0.0).astype(f32)
  dn = (((0,), (0,)), ((), ()))
  acc_s[...] += lax.dot_general(oh, node, dn, preferred_element_type=f32)
  ones = jnp.ones((BN, MSG), f32)
  acc_c[...] += lax.dot_general(oh, ones, dn, preferred_element_type=f32)

  @pl.when(i == NBLK - 1)
  def _():
    pooled = acc_s[...] / jnp.maximum(acc_c[...], 1.0)
    out_ref[...] = (jnp.dot(pooled, lw[...], preferred_element_type=f32)
                    + blr[...])


def _tc_node(parts, batch_i, w1, b1, w2, b2, w3, b3, w4, b4, lw, bl):
  ws = lambda shape: pl.BlockSpec(shape, lambda i: (0, 0))
  return pl.pallas_call(
      _node_body,
      grid=(NBLK,),
      in_specs=[
          pl.BlockSpec((NC, BN, MSG), lambda i: (0, i, 0)),
          pl.BlockSpec((BN, 1), lambda i: (i, 0)),
          ws((MSG, H)), ws((1, H)),
          ws((H, H)), ws((1, H)),
          ws((H, H)), ws((1, H)),
          ws((H, NH)), ws((1, NH)),
          ws((NH, NP)), ws((1, NP)),
      ],
      out_specs=pl.BlockSpec((G, NP), lambda i: (0, 0)),
      out_shape=jax.ShapeDtypeStruct((G, NP), jnp.float32),
      scratch_shapes=[
          pltpu.VMEM((G, NH), jnp.float32),
          pltpu.VMEM((G, NH), jnp.float32),
      ],
      compiler_params=pltpu.CompilerParams(
          dimension_semantics=("arbitrary",)),
  )(parts, batch_i, w1, b1, w2, b2, w3, b3, w4, b4, lw, bl)


def kernel(x, edge_index, edge_attr, batch,
           mW1, mb1, mW2, mb2, mW3, mb3, mW4, mb4,
           nW1, nb1, nW2, nb2, nW3, nb3, nW4, nb4,
           L, bL):
  padidx = jnp.arange(EPAD - E, dtype=jnp.int32) % N
  src3 = jnp.concatenate([edge_index[0], padidx]).reshape(NW, CPW, ECH)
  dst3g = jnp.concatenate([edge_index[1], padidx]).reshape(NW, CPW, ECH)
  dst3s = jnp.pad(edge_index[1], ((0, EPAD - E),),
                  constant_values=DUMP).reshape(NW, CPW, ECH)
  ea_p = jnp.pad(edge_attr, ((0, EPAD - E), (0, 0)))

  xi, xj = _sc_gather(x, src3, dst3g)

  w1a = mW1[:NF]
  w1b = mW1[NF:2 * NF]
  w1c = mW1[2 * NF:]
  msg = _tc_edge_mlp(xi, xj, ea_p,
                     w1a, w1b, w1c, mb1.reshape(1, H),
                     mW2, mb2.reshape(1, H),
                     mW3, mb3.reshape(1, H),
                     mW4, mb4.reshape(1, MSG))

  zero = jnp.zeros((NPAD, MSG), jnp.float32)
  parts = _sc_scatter(msg, dst3s, zero)

  batch_i = batch.reshape(N, 1)
  out = _tc_node(parts, batch_i,
                 nW1, nb1.reshape(1, H),
                 nW2, nb2.reshape(1, H),
                 nW3, nb3.reshape(1, H),
                 nW4, nb4.reshape(1, NH),
                 L, bL.reshape(1, NP))
  return out


# speedup vs baseline: 1.9903x; 1.1560x over previous
import jax
import jax.numpy as jnp
from jax import lax
from jax.experimental import pallas as pl
from jax.experimental.pallas import tpu as pltpu
from jax.experimental.pallas import tpu_sc as plsc

N = 10000
E = 320000
NF = 128
EF = 16
MSG = 128
H = 300
NH = 128
NP = 32
G = 64

NC = 2
NS = 16
NW = NC * NS

S = 4
ESL = E // S
ECH = 128
CPW = 20
EPAD = NW * CPW * ECH

NPAD = 10240
STRIPE = NPAD // NS
DUMP = NPAD - 1

_MESH = dict(core_axis_name="c", subcore_axis_name="s", num_cores=NC,
             num_subcores=NS)


def _gather_body(x_hbm, src_hbm, dst_hbm, xi_hbm, xj_hbm,
                 idx_s, idx_d, bi0, bi1, bj0, bj1, sem):
  wid = lax.axis_index("s") * NC + lax.axis_index("c")
  pltpu.sync_copy(src_hbm.at[wid], idx_s)
  pltpu.sync_copy(dst_hbm.at[wid], idx_d)
  base = wid * CPW

  def step(m, carry):
    r0 = 2 * m
    r1 = 2 * m + 1
    c0 = pltpu.async_copy(x_hbm.at[idx_d.at[r0]], bi0, sem)
    c1 = pltpu.async_copy(x_hbm.at[idx_s.at[r0]], bj0, sem)
    c2 = pltpu.async_copy(x_hbm.at[idx_d.at[r1]], bi1, sem)
    c3 = pltpu.async_copy(x_hbm.at[idx_s.at[r1]], bj1, sem)
    c0.wait()
    c1.wait()
    c2.wait()
    c3.wait()
    off = (base + r0) * ECH
    pltpu.sync_copy(bi0, xi_hbm.at[pl.ds(off, ECH)])
    pltpu.sync_copy(bj0, xj_hbm.at[pl.ds(off, ECH)])
    pltpu.sync_copy(bi1, xi_hbm.at[pl.ds(off + ECH, ECH)])
    pltpu.sync_copy(bj1, xj_hbm.at[pl.ds(off + ECH, ECH)])
    return carry

  lax.fori_loop(0, CPW // 2, step, 0)


def _sc_gather(x, src3, dst3):
  mesh = plsc.VectorSubcoreMesh(**_MESH)
  buf = pltpu.VMEM((ECH, NF), jnp.float32)
  fn = pl.kernel(
      _gather_body,
      out_type=(jax.ShapeDtypeStruct((EPAD, NF), jnp.float32),
                jax.ShapeDtypeStruct((EPAD, NF), jnp.float32)),
      mesh=mesh,
      scratch_types=[
          pltpu.VMEM((CPW, ECH), jnp.int32),
          pltpu.VMEM((CPW, ECH), jnp.int32),
          buf, buf, buf, buf,
          pltpu.SemaphoreType.DMA,
      ],
  )
  return fn(x, src3, dst3)


def _scatter_body(msg_hbm, dst_hbm, zero_hbm, out_hbm,
                  idx_d, mb0, mb1, acc, sem_m):
  cid = lax.axis_index("c")
  sid = lax.axis_index("s")
  wid = sid * NC + cid
  pltpu.sync_copy(zero_hbm.at[pl.ds(sid * STRIPE, STRIPE)],
                  acc.at[pl.ds(sid * STRIPE, STRIPE)])
  plsc.subcore_barrier()

  pltpu.sync_copy(dst_hbm.at[wid], idx_d)
  mbs = [mb0, mb1]
  base = wid * CPW

  def load(r, slot):
    off = (base + r) * ECH
    pltpu.async_copy(msg_hbm.at[pl.ds(off, ECH)], mbs[slot], sem_m)

  def wait_load(slot):
    pltpu.make_async_copy(msg_hbm.at[pl.ds(0, ECH)], mbs[slot], sem_m).wait()

  load(0, 0)

  def step(m, carry):
    for b in range(2):
      r = 2 * m + b

      @pl.when(r < CPW)
      def _():
        @pl.when(r + 1 < CPW)
        def _():
          load(r + 1, (b + 1) % 2)

        wait_load(b)
        pltpu.sync_copy(mbs[b], acc.at[idx_d.at[r]], add=True)
    return carry

  lax.fori_loop(0, (CPW + 1) // 2, step, 0)

  plsc.subcore_barrier()
  pltpu.sync_copy(acc.at[pl.ds(sid * STRIPE, STRIPE)],
                  out_hbm.at[cid, pl.ds(sid * STRIPE, STRIPE)])


def _sc_scatter(msg, dst3, zero):
  mesh = plsc.VectorSubcoreMesh(**_MESH)
  fn = pl.kernel(
      _scatter_body,
      out_type=jax.ShapeDtypeStruct((NC, NPAD, MSG), jnp.float32),
      mesh=mesh,
      scratch_types=[
          pltpu.VMEM((CPW, ECH), jnp.int32),
          pltpu.VMEM((ECH, MSG), jnp.float32),
          pltpu.VMEM((ECH, MSG), jnp.float32),
          pltpu.VMEM_SHARED((NPAD, MSG), jnp.float32),
          pltpu.SemaphoreType.DMA,
      ],
  )
  return fn(msg, dst3, zero)


BE = 4096
NEB = EPAD // BE


def _emlp_body(xi_ref, xj_ref, ea_ref, w1a, w1b, w1c, b1, w2, b2, w3, b3,
               w4, b4, out_ref):
  f32 = jnp.float32
  h = jnp.dot(xi_ref[...], w1a[...], preferred_element_type=f32)
  h = h + jnp.dot(xj_ref[...], w1b[...], preferred_element_type=f32)
  h = h + jnp.dot(ea_ref[...], w1c[...], preferred_element_type=f32)
  h = jnp.maximum(h + b1[...], 0.0)
  h = jnp.maximum(jnp.dot(h, w2[...], preferred_element_type=f32) + b2[...],
                  0.0)
  h = jnp.maximum(jnp.dot(h, w3[...], preferred_element_type=f32) + b3[...],
                  0.0)
  out_ref[...] = jnp.dot(h, w4[...], preferred_element_type=f32) + b4[...]


def _tc_edge_mlp(xi, xj, ea, w1a, w1b, w1c, b1, w2, b2, w3, b3, w4, b4):
  ws = lambda shape: pl.BlockSpec(shape, lambda i: (0, 0))
  return pl.pallas_call(
      _emlp_body,
      grid=(NEB,),
      in_specs=[
          pl.BlockSpec((BE, NF), lambda i: (i, 0)),
          pl.BlockSpec((BE, NF), lambda i: (i, 0)),
          pl.BlockSpec((BE, EF), lambda i: (i, 0)),
          ws((NF, H)), ws((NF, H)), ws((EF, H)), ws((1, H)),
          ws((H, H)), ws((1, H)),
          ws((H, H)), ws((1, H)),
          ws((H, MSG)), ws((1, MSG)),
      ],
      out_specs=pl.BlockSpec((BE, MSG), lambda i: (i, 0)),
      out_shape=jax.ShapeDtypeStruct((EPAD, MSG), jnp.float32),
      compiler_params=pltpu.CompilerParams(
          dimension_semantics=("arbitrary",)),
  )(xi, xj, ea, w1a, w1b, w1c, b1, w2, b2, w3, b3, w4, b4)


BN = 2000
NBLK = N // BN


def _node_body(p0_ref, p1_ref, p2_ref, p3_ref, batch_ref,
               w1, b1, w2, b2, w3, b3, w4, b4,
               lw, blr, out_ref, acc_s, acc_c):
  f32 = jnp.float32
  i = pl.program_id(0)

  @pl.when(i == 0)
  def _():
    acc_s[...] = jnp.zeros_like(acc_s)
    acc_c[...] = jnp.zeros_like(acc_c)

  aggr = ((p0_ref[0] + p0_ref[1]) + (p1_ref[0] + p1_ref[1])
          + (p2_ref[0] + p2_ref[1]) + (p3_ref[0] + p3_ref[1]))
  h = jnp.maximum(jnp.dot(aggr, w1[...], preferred_element_type=f32)
                  + b1[...], 0.0)
  h = jnp.maximum(jnp.dot(h, w2[...], preferred_element_type=f32) + b2[...],
                  0.0)
  h = jnp.maximum(jnp.dot(h, w3[...], preferred_element_type=f32) + b3[...],
                  0.0)
  node = jnp.dot(h, w4[...], preferred_element_type=f32) + b4[...]

  gid = lax.broadcasted_iota(jnp.int32, (BN, G), 1)
  oh = jnp.where(batch_ref[...] == gid, 1.0, 0.0).astype(f32)
  dn = (((0,), (0,)), ((), ()))
  acc_s[...] += lax.dot_general(oh, node, dn, preferred_element_type=f32)
  ones = jnp.ones((BN, MSG), f32)
  acc_c[...] += lax.dot_general(oh, ones, dn, preferred_element_type=f32)

  @pl.when(i == NBLK - 1)
  def _():
    pooled = acc_s[...] / jnp.maximum(acc_c[...], 1.0)
    out_ref[...] = (jnp.dot(pooled, lw[...], preferred_element_type=f32)
                    + blr[...])


def _tc_node(parts, batch_i, w1, b1, w2, b2, w3, b3, w4, b4, lw, bl):
  ws = lambda shape: pl.BlockSpec(shape, lambda i: (0, 0))
  pspec = pl.BlockSpec((NC, BN, MSG), lambda i: (0, i, 0))
  return pl.pallas_call(
      _node_body,
      grid=(NBLK,),
      in_specs=[
          pspec, pspec, pspec, pspec,
          pl.BlockSpec((BN, 1), lambda i: (i, 0)),
          ws((MSG, H)), ws((1, H)),
          ws((H, H)), ws((1, H)),
          ws((H, H)), ws((1, H)),
          ws((H, NH)), ws((1, NH)),
          ws((NH, NP)), ws((1, NP)),
      ],
      out_specs=pl.BlockSpec((G, NP), lambda i: (0, 0)),
      out_shape=jax.ShapeDtypeStruct((G, NP), jnp.float32),
      scratch_shapes=[
          pltpu.VMEM((G, NH), jnp.float32),
          pltpu.VMEM((G, NH), jnp.float32),
      ],
      compiler_params=pltpu.CompilerParams(
          dimension_semantics=("arbitrary",)),
  )(*parts, batch_i, w1, b1, w2, b2, w3, b3, w4, b4, lw, bl)


def kernel(x, edge_index, edge_attr, batch,
           mW1, mb1, mW2, mb2, mW3, mb3, mW4, mb4,
           nW1, nb1, nW2, nb2, nW3, nb3, nW4, nb4,
           L, bL):
  padidx = jnp.arange(EPAD - ESL, dtype=jnp.int32) % N
  srcs = edge_index[0].reshape(S, ESL)
  dsts = edge_index[1].reshape(S, ESL)

  w1a = mW1[:NF]
  w1b = mW1[NF:2 * NF]
  w1c = mW1[2 * NF:]
  mb1r = mb1.reshape(1, H)
  mb2r = mb2.reshape(1, H)
  mb3r = mb3.reshape(1, H)
  mb4r = mb4.reshape(1, MSG)
  zero = jnp.zeros((NPAD, MSG), jnp.float32)
  ea_s = jnp.pad(edge_attr.reshape(S, ESL, EF),
                 ((0, 0), (0, EPAD - ESL), (0, 0)))

  parts = []
  for s in range(S):
    src3 = jnp.concatenate([srcs[s], padidx]).reshape(NW, CPW, ECH)
    dst3g = jnp.concatenate([dsts[s], padidx]).reshape(NW, CPW, ECH)
    dst3s = jnp.pad(dsts[s], ((0, EPAD - ESL),),
                    constant_values=DUMP).reshape(NW, CPW, ECH)
    xi, xj = _sc_gather(x, src3, dst3g)
    msg = _tc_edge_mlp(xi, xj, ea_s[s],
                       w1a, w1b, w1c, mb1r, mW2, mb2r, mW3, mb3r, mW4, mb4r)
    parts.append(_sc_scatter(msg, dst3s, zero))

  batch_i = batch.reshape(N, 1)
  out = _tc_node(parts, batch_i,
                 nW1, nb1.reshape(1, H),
                 nW2, nb2.reshape(1, H),
                 nW3, nb3.reshape(1, H),
                 nW4, nb4.reshape(1, NH),
                 L, bL.reshape(1, NP))
  return out
